# baseline (speedup 1.0000x reference)
import jax
import jax.numpy as jnp
from jax.experimental import pallas as pl
from jax.experimental.pallas import tpu as pltpu

Q = 5.0
WARMUP = 200
MINK = 100

H = 128
W = 128
B = 16
N_TOTAL = B * H * W
K_TOP = max(MINK, int(N_TOTAL * Q / 100.0))
CB = 48
ROWS = N_TOTAL // W

_INT_MIN = -2147483648
_INT_MAXP = 0x7FFFFFFF


def _partials(en_ref, de_ref):
    a = en_ref[0]
    b = de_ref[0]
    return jnp.sum(a * a, axis=0), jnp.sum(a * b, axis=0), jnp.sum(b * b, axis=0)


def _fused_body(en0_ref, de0_ref, en1_ref, de1_ref, en2_ref, de2_ref,
                out_ref, aa, ab, bb, dacc, v_ref, s_ref):
    i = pl.program_id(0)
    j = pl.program_id(1)

    def accumulate(en_ref, de_ref, init, fin, first_scale):
        paa, pab, pbb = _partials(en_ref, de_ref)

        @pl.when(init)
        def _():
            aa[...] = paa
            ab[...] = pab
            bb[...] = pbb

        @pl.when(jnp.logical_not(init))
        def _():
            aa[...] += paa
            ab[...] += pab
            bb[...] += pbb

        @pl.when(fin)
        def _():
            na = jnp.maximum(jnp.sqrt(aa[...]), 1e-8)
            nb = jnp.maximum(jnp.sqrt(bb[...]), 1e-8)
            d = 1.0 - ab[...] / (na * nb)
            if first_scale:
                dacc[...] = d
            else:
                dacc[...] += d

    @pl.when(j <= 1)
    def _():
        accumulate(en0_ref, de0_ref, j == 0, j == 1, True)

    @pl.when((j >= 2) & (j <= 5))
    def _():
        accumulate(en1_ref, de1_ref, j == 2, j == 5, False)

    @pl.when(j >= 6)
    def _():
        accumulate(en2_ref, de2_ref, j == 6, j == 13, False)

    @pl.when(j == 13)
    def _():
        v = dacc[...] * (1.0 / 3.0)
        rows = pl.ds(i * H, H)
        v_ref[rows, :] = v
        bits = jax.lax.bitcast_convert_type(v, jnp.int32)
        s_ref[rows, :] = bits ^ (
            jax.lax.shift_right_arithmetic(bits, 31) & jnp.int32(_INT_MAXP)
        )

    @pl.when((i == B - 1) & (j == 13))
    def _():
        kk = jnp.int32(K_TOP)

        def body(it, prefix_u):
            bit = jax.lax.shift_left(jnp.int32(1), jnp.int32(31) - it)
            cand_u = prefix_u | bit
            thresh_s = cand_u ^ jnp.int32(_INT_MIN)
            cnt = jnp.sum((s_ref[...] >= thresh_s).astype(jnp.int32))
            return jnp.where(cnt >= kk, cand_u, prefix_u)

        prefix_u = jax.lax.fori_loop(0, 32, body, jnp.int32(0))
        t_s = prefix_u ^ jnp.int32(_INT_MIN)
        t_bits = jnp.where(t_s >= 0, t_s, t_s ^ jnp.int32(_INT_MAXP))
        t_val = jax.lax.bitcast_convert_type(t_bits, jnp.float32)

        sdat = s_ref[...]
        gt = sdat > t_s
        cnt_gt = jnp.sum(gt.astype(jnp.float32))
        sum_gt = jnp.sum(jnp.where(gt, v_ref[...], 0.0))
        out_ref[0, 0] = (sum_gt + (jnp.float32(K_TOP) - cnt_gt) * t_val) * (
            1.0 / K_TOP
        )


def _fused(en0, de0, en1, de1, en2, de2):
    def blk(nblocks, start):
        return pl.BlockSpec(
            (1, CB, H, W),
            lambda i, j: (i, jnp.clip(j - start, 0, nblocks - 1), 0, 0),
        )

    s0 = blk(2, 0)
    s1 = blk(4, 2)
    s2 = blk(8, 6)
    return pl.pallas_call(
        _fused_body,
        grid=(B, 14),
        in_specs=[s0, s0, s1, s1, s2, s2],
        out_specs=pl.BlockSpec(memory_space=pltpu.SMEM),
        out_shape=jax.ShapeDtypeStruct((1, 1), jnp.float32),
        scratch_shapes=[
            pltpu.VMEM((H, W), jnp.float32),
            pltpu.VMEM((H, W), jnp.float32),
            pltpu.VMEM((H, W), jnp.float32),
            pltpu.VMEM((H, W), jnp.float32),
            pltpu.VMEM((ROWS, W), jnp.float32),
            pltpu.VMEM((ROWS, W), jnp.int32),
        ],
    )(en0, de0, en1, de1, en2, de2)


def kernel(en0, en1, en2, de0, de1, de2, global_step):
    topk_mean = _fused(en0, de0, en1, de1, en2, de2)[0, 0]
    progress = global_step / WARMUP
    warm = 100.0 - (100.0 - Q) * progress
    q_current = jnp.where(global_step < WARMUP, warm, Q).astype(jnp.float32)
    return topk_mean * (q_current / Q)


# speedup vs baseline: 1.0862x; 1.0862x over previous
import jax
import jax.numpy as jnp
from jax.experimental import pallas as pl
from jax.experimental.pallas import tpu as pltpu

Q = 5.0
WARMUP = 200
MINK = 100

H = 128
W = 128
B = 16
N_TOTAL = B * H * W
K_TOP = max(MINK, int(N_TOTAL * Q / 100.0))
CB = 48
ROWS = N_TOTAL // W

_INT_MIN = -2147483648
_INT_MAXP = 0x7FFFFFFF


def _partials(en_ref, de_ref):
    a = en_ref[0]
    b = de_ref[0]
    return jnp.sum(a * a, axis=0), jnp.sum(a * b, axis=0), jnp.sum(b * b, axis=0)


def _fused_body(en0_ref, de0_ref, en1_ref, de1_ref, en2_ref, de2_ref,
                out_ref, aa, ab, bb, dacc, v_ref, s_ref):
    i = pl.program_id(0)
    j = pl.program_id(1)

    def accumulate(en_ref, de_ref, init, fin, first_scale):
        paa, pab, pbb = _partials(en_ref, de_ref)

        @pl.when(init)
        def _():
            aa[...] = paa
            ab[...] = pab
            bb[...] = pbb

        @pl.when(jnp.logical_not(init))
        def _():
            aa[...] += paa
            ab[...] += pab
            bb[...] += pbb

        @pl.when(fin)
        def _():
            na = jnp.maximum(jnp.sqrt(aa[...]), 1e-8)
            nb = jnp.maximum(jnp.sqrt(bb[...]), 1e-8)
            d = 1.0 - ab[...] / (na * nb)
            if first_scale:
                dacc[...] = d
            else:
                dacc[...] += d

    @pl.when(j <= 1)
    def _():
        accumulate(en0_ref, de0_ref, j == 0, j == 1, True)

    @pl.when((j >= 2) & (j <= 5))
    def _():
        accumulate(en1_ref, de1_ref, j == 2, j == 5, False)

    @pl.when(j >= 6)
    def _():
        accumulate(en2_ref, de2_ref, j == 6, j == 13, False)

    @pl.when(j == 13)
    def _():
        v = dacc[...] * (1.0 / 3.0)
        rows = pl.ds(i * H, H)
        v_ref[rows, :] = v
        bits = jax.lax.bitcast_convert_type(v, jnp.int32)
        s_ref[rows, :] = bits ^ (
            jax.lax.shift_right_arithmetic(bits, 31) & jnp.int32(_INT_MAXP)
        )

    @pl.when((i == B - 1) & (j == 13))
    def _():
        kk = jnp.int32(K_TOP)

        def body(it, prefix_u):
            bit = jax.lax.shift_left(jnp.int32(1), jnp.int32(31) - it)
            cand_u = prefix_u | bit
            thresh_s = cand_u ^ jnp.int32(_INT_MIN)
            cnt = jnp.sum((s_ref[...] >= thresh_s).astype(jnp.int32))
            return jnp.where(cnt >= kk, cand_u, prefix_u)

        prefix_u = jax.lax.fori_loop(0, 32, body, jnp.int32(0))
        t_s = prefix_u ^ jnp.int32(_INT_MIN)
        t_bits = jnp.where(t_s >= 0, t_s, t_s ^ jnp.int32(_INT_MAXP))
        t_val = jax.lax.bitcast_convert_type(t_bits, jnp.float32)

        sdat = s_ref[...]
        gt = sdat > t_s
        cnt_gt = jnp.sum(gt.astype(jnp.float32))
        sum_gt = jnp.sum(jnp.where(gt, v_ref[...], 0.0))
        out_ref[0, 0] = (sum_gt + (jnp.float32(K_TOP) - cnt_gt) * t_val) * (
            1.0 / K_TOP
        )


def _fused(en0, de0, en1, de1, en2, de2):
    def blk(nblocks, start):
        def index_map(i, j):
            active = j >= start
            ii = jnp.where(active, i, jnp.maximum(i - 1, 0))
            bi = jnp.where(
                active,
                jnp.clip(j - start, 0, nblocks - 1),
                jnp.where(i == 0, 0, nblocks - 1),
            )
            return (ii, bi, 0, 0)

        return pl.BlockSpec((1, CB, H, W), index_map)

    s0 = blk(2, 0)
    s1 = blk(4, 2)
    s2 = blk(8, 6)
    return pl.pallas_call(
        _fused_body,
        grid=(B, 14),
        in_specs=[s0, s0, s1, s1, s2, s2],
        out_specs=pl.BlockSpec(memory_space=pltpu.SMEM),
        out_shape=jax.ShapeDtypeStruct((1, 1), jnp.float32),
        scratch_shapes=[
            pltpu.VMEM((H, W), jnp.float32),
            pltpu.VMEM((H, W), jnp.float32),
            pltpu.VMEM((H, W), jnp.float32),
            pltpu.VMEM((H, W), jnp.float32),
            pltpu.VMEM((ROWS, W), jnp.float32),
            pltpu.VMEM((ROWS, W), jnp.int32),
        ],
    )(en0, de0, en1, de1, en2, de2)


def kernel(en0, en1, en2, de0, de1, de2, global_step):
    topk_mean = _fused(en0, de0, en1, de1, en2, de2)[0, 0]
    progress = global_step / WARMUP
    warm = 100.0 - (100.0 - Q) * progress
    q_current = jnp.where(global_step < WARMUP, warm, Q).astype(jnp.float32)
    return topk_mean * (q_current / Q)
